# baseline (speedup 1.0000x reference)
import functools

import jax
import jax.numpy as jnp
from jax import lax
from jax.experimental import pallas as pl
from jax.experimental.pallas import tpu as pltpu
from jax.experimental.pallas import tpu_sc as plsc

_NUM_CORES = 2
_NUM_SUBCORES = 16
_NW = _NUM_CORES * _NUM_SUBCORES

_B = 4096
_D = 128
_BPW = _B // _NW


def _sc_body(idx_hbm, mask_hbm, out_hbm, idx_v, rows_v, sem):
    wid = lax.axis_index("s") * _NUM_CORES + lax.axis_index("c")
    base = wid * _BPW
    pltpu.sync_copy(idx_hbm.at[pl.ds(base, _BPW)], idx_v)
    pltpu.async_copy(mask_hbm.at[idx_v], rows_v, sem).wait()
    pltpu.sync_copy(rows_v, out_hbm.at[pl.ds(base, _BPW)])


_sc_expand = functools.partial(
    pl.kernel,
    out_type=jax.ShapeDtypeStruct((_B, _D), jnp.float32),
    mesh=plsc.VectorSubcoreMesh(core_axis_name="c", subcore_axis_name="s"),
    scratch_types=[
        pltpu.VMEM((_BPW,), jnp.int32),
        pltpu.VMEM((_BPW, _D), jnp.float32),
        pltpu.SemaphoreType.DMA,
    ],
)(_sc_body)


def kernel(inps, times, mask):
    del inps
    idx = jnp.full((_B,), times, dtype=jnp.int32)
    return _sc_expand(idx, mask)


# speedup vs baseline: 5.7711x; 5.7711x over previous
import functools

import jax
import jax.numpy as jnp
from jax import lax
from jax.experimental import pallas as pl
from jax.experimental.pallas import tpu as pltpu
from jax.experimental.pallas import tpu_sc as plsc

_NUM_CORES = 2
_NUM_SUBCORES = 16
_NW = _NUM_CORES * _NUM_SUBCORES

_B = 4096
_D = 128
_BPW = _B // _NW
_L = 16
_GR = 8


def _sc_body(idx_hbm, mask_hbm, out_hbm, idx_v, buf_v, sem):
    wid = lax.axis_index("s") * _NUM_CORES + lax.axis_index("c")
    base = wid * _BPW
    pltpu.sync_copy(idx_hbm.at[pl.ds(wid * _GR, _GR)], idx_v)
    pltpu.async_copy(mask_hbm.at[idx_v], buf_v.at[pl.ds(0, _GR)], sem).wait()
    vals = [buf_v[0, pl.ds(j * _L, _L)] for j in range(_D // _L)]

    @pl.loop(_GR, _BPW)
    def _(r):
        for j in range(_D // _L):
            buf_v[r, pl.ds(j * _L, _L)] = vals[j]

    pltpu.sync_copy(buf_v, out_hbm.at[pl.ds(base, _BPW)])


_sc_expand = functools.partial(
    pl.kernel,
    out_type=jax.ShapeDtypeStruct((_B, _D), jnp.float32),
    mesh=plsc.VectorSubcoreMesh(core_axis_name="c", subcore_axis_name="s"),
    scratch_types=[
        pltpu.VMEM((_GR,), jnp.int32),
        pltpu.VMEM((_BPW, _D), jnp.float32),
        pltpu.SemaphoreType.DMA,
    ],
)(_sc_body)


def kernel(inps, times, mask):
    del inps
    idx = jnp.full((_NW * _GR,), times, dtype=jnp.int32)
    return _sc_expand(idx, mask)
